# Initial kernel scaffold; baseline (speedup 1.0000x reference)
import jax
import jax.numpy as jnp
from jax.experimental import pallas as pl


def kernel(t, x, ts, us):
    raise NotImplementedError("write your pallas kernel here")


# speedup vs baseline: 7.5662x; 7.5662x over previous
import functools

import jax
import jax.numpy as jnp
from jax import lax
from jax.experimental import pallas as pl
from jax.experimental.pallas import tpu as pltpu
from jax.experimental.pallas import tpu_sc as plsc

N_SEG = 1_000_000
U_DIM = 8
N_QUERIES = 1_048_576
ROW = 16
M = N_SEG // ROW
NC, NS, L = 2, 16, 16
NW = NC * NS
QW = N_QUERIES // NW
B = 1024
NB = QW // B
CH = 128
NCH = B // CH

_COARSE_STEPS = 16
_FINE_STEPS = 5


def _body(t_hbm, c_hbm, ts2_hbm, us_hbm, out_hbm,
          c_v, tq, ridx, rows, fidx, urows, sem):
    wid = lax.axis_index("s") * NC + lax.axis_index("c")
    pltpu.sync_copy(c_hbm, c_v)
    lanes = lax.iota(jnp.int32, L)

    def batch_body(b, _):
        qoff = wid * QW + b * B
        pltpu.sync_copy(t_hbm.at[pl.ds(qoff, B)], tq)

        def vec_coarse(v, _):
            tv = tq[pl.ds(v * L, L)]
            lo = jnp.zeros((L,), jnp.int32)
            hi = jnp.full((L,), M, jnp.int32)

            def step(i, carry):
                lo, hi = carry
                mid = jnp.minimum((lo + hi) >> 1, M - 1)
                le = plsc.load_gather(c_v, [mid]) <= tv
                return jnp.where(le, mid + 1, lo), jnp.where(le, hi, mid)

            lo, hi = lax.fori_loop(0, _COARSE_STEPS, step, (lo, hi))
            ridx[pl.ds(v * L, L)] = jnp.maximum(lo - 1, 0)
            return 0

        lax.fori_loop(0, B // L, vec_coarse, 0)

        descs = [
            pltpu.async_copy(ts2_hbm.at[ridx.at[pl.ds(j * CH, CH)]],
                             rows.at[pl.ds(j * CH, CH)], sem)
            for j in range(NCH)
        ]
        for d in descs:
            d.wait()

        def vec_fine(v, _):
            tv = tq[pl.ds(v * L, L)]
            qv = v * L + lanes
            lo = jnp.zeros((L,), jnp.int32)
            hi = jnp.full((L,), ROW, jnp.int32)

            def step(i, carry):
                lo, hi = carry
                mid = jnp.minimum((lo + hi) >> 1, ROW - 1)
                le = plsc.load_gather(rows, [qv, mid]) <= tv
                return jnp.where(le, mid + 1, lo), jnp.where(le, hi, mid)

            lo, hi = lax.fori_loop(0, _FINE_STEPS, step, (lo, hi))
            rc = ridx[pl.ds(v * L, L)]
            gidx = rc * ROW + lo - 1
            fidx[pl.ds(v * L, L)] = jnp.clip(gidx, 0, N_SEG - 1)
            return 0

        lax.fori_loop(0, B // L, vec_fine, 0)

        descs = [
            pltpu.async_copy(us_hbm.at[fidx.at[pl.ds(j * CH, CH)]],
                             urows.at[pl.ds(j * CH, CH)], sem)
            for j in range(NCH)
        ]
        for d in descs:
            d.wait()
        pltpu.sync_copy(urows, out_hbm.at[pl.ds(qoff, B)])
        return 0

    lax.fori_loop(0, NB, batch_body, 0)


@jax.jit
def _run(t, c, ts2, us):
    mesh = plsc.VectorSubcoreMesh(core_axis_name="c", subcore_axis_name="s")
    return pl.kernel(
        _body,
        out_type=jax.ShapeDtypeStruct((N_QUERIES, U_DIM), jnp.float32),
        mesh=mesh,
        compiler_params=pltpu.CompilerParams(
            needs_layout_passes=False, use_tc_tiling_on_sc=False),
        scratch_types=[
            pltpu.VMEM((M,), jnp.float32),
            pltpu.VMEM((B,), jnp.float32),
            pltpu.VMEM((B,), jnp.int32),
            pltpu.VMEM((B, ROW), jnp.float32),
            pltpu.VMEM((B,), jnp.int32),
            pltpu.VMEM((B, U_DIM), jnp.float32),
            pltpu.SemaphoreType.DMA,
        ],
    )(t, c, ts2, us)


def kernel(t, x, ts, us):
    c = ts[::ROW]
    ts2 = ts.reshape(M, ROW)
    return _run(t, c, ts2, us)


# speedup vs baseline: 9.8668x; 1.3041x over previous
import functools

import jax
import jax.numpy as jnp
from jax import lax
from jax.experimental import pallas as pl
from jax.experimental.pallas import tpu as pltpu
from jax.experimental.pallas import tpu_sc as plsc

N_SEG = 1_000_000
U_DIM = 8
N_QUERIES = 1_048_576
ROW = 16
M = N_SEG // ROW
U2 = N_SEG * U_DIM // 16
NC, NS, L = 2, 16, 16
NW = NC * NS
QW = N_QUERIES // NW
B = 1024
NB = QW // B
CH = 128
NCH = B // CH

_COARSE_STEPS = 16
_FINE_STEPS = 5


def _body(t_hbm, c_hbm, ts2_hbm, us16_hbm, out_hbm,
          c_v, tq, ridx, rows, gidx, par, urows, obuf, sem):
    wid = lax.axis_index("s") * NC + lax.axis_index("c")
    pltpu.sync_copy(c_hbm, c_v)
    lanes = lax.iota(jnp.int32, L)

    def batch_body(b, _):
        qoff = wid * QW + b * B
        pltpu.sync_copy(t_hbm.at[pl.ds(qoff, B)], tq)

        def vec_coarse(v, _):
            tv = tq[pl.ds(v * L, L)]
            lo = jnp.zeros((L,), jnp.int32)
            hi = jnp.full((L,), M, jnp.int32)

            def step(i, carry):
                lo, hi = carry
                mid = jnp.minimum((lo + hi) >> 1, M - 1)
                le = plsc.load_gather(c_v, [mid]) <= tv
                return jnp.where(le, mid + 1, lo), jnp.where(le, hi, mid)

            lo, hi = lax.fori_loop(0, _COARSE_STEPS, step, (lo, hi))
            ridx[pl.ds(v * L, L)] = jnp.maximum(lo - 1, 0)
            return 0

        lax.fori_loop(0, B // L, vec_coarse, 0)

        descs = [
            pltpu.async_copy(ts2_hbm.at[ridx.at[pl.ds(j * CH, CH)]],
                             rows.at[pl.ds(j * CH, CH)], sem)
            for j in range(NCH)
        ]
        for d in descs:
            d.wait()

        def vec_fine(v, _):
            tv = tq[pl.ds(v * L, L)]
            qv = v * L + lanes
            lo = jnp.zeros((L,), jnp.int32)
            hi = jnp.full((L,), ROW, jnp.int32)

            def step(i, carry):
                lo, hi = carry
                mid = jnp.minimum((lo + hi) >> 1, ROW - 1)
                le = plsc.load_gather(rows, [qv, mid]) <= tv
                return jnp.where(le, mid + 1, lo), jnp.where(le, hi, mid)

            lo, hi = lax.fori_loop(0, _FINE_STEPS, step, (lo, hi))
            rc = ridx[pl.ds(v * L, L)]
            fi = jnp.clip(rc * ROW + lo - 1, 0, N_SEG - 1)
            gidx[pl.ds(v * L, L)] = fi >> 1
            par[pl.ds(v * L, L)] = (fi & 1) << 3
            return 0

        lax.fori_loop(0, B // L, vec_fine, 0)

        descs = [
            pltpu.async_copy(us16_hbm.at[gidx.at[pl.ds(j * CH, CH)]],
                             urows.at[pl.ds(j * CH, CH)], sem)
            for j in range(NCH)
        ]
        for d in descs:
            d.wait()

        def vec_out(v, _):
            qv = v * L + lanes
            pv = par[pl.ds(v * L, L)]
            base = (v >> 3) * 1024 + (v & 7) * L
            for j in range(U_DIM):
                val = plsc.load_gather(urows, [qv, pv + j])
                obuf[pl.ds(base + j * 128, L)] = val
            return 0

        lax.fori_loop(0, B // L, vec_out, 0)
        pltpu.sync_copy(obuf, out_hbm.at[pl.ds(qoff * U_DIM, B * U_DIM)])
        return 0

    lax.fori_loop(0, NB, batch_body, 0)


@jax.jit
def _run(t, c, ts2, us16):
    mesh = plsc.VectorSubcoreMesh(core_axis_name="c", subcore_axis_name="s")
    return pl.kernel(
        _body,
        out_type=jax.ShapeDtypeStruct((N_QUERIES * U_DIM,), jnp.float32),
        mesh=mesh,
        compiler_params=pltpu.CompilerParams(
            needs_layout_passes=False, use_tc_tiling_on_sc=False),
        scratch_types=[
            pltpu.VMEM((M,), jnp.float32),
            pltpu.VMEM((B,), jnp.float32),
            pltpu.VMEM((B,), jnp.int32),
            pltpu.VMEM((B, ROW), jnp.float32),
            pltpu.VMEM((B,), jnp.int32),
            pltpu.VMEM((B,), jnp.int32),
            pltpu.VMEM((B, 16), jnp.float32),
            pltpu.VMEM((B * U_DIM,), jnp.float32),
            pltpu.SemaphoreType.DMA,
        ],
    )(t, c, ts2, us16)


def kernel(t, x, ts, us):
    c = ts[::ROW]
    ts2 = ts.reshape(M, ROW)
    us16 = us.reshape(U2, 16)
    o = _run(t, c, ts2, us16)
    o3 = o.reshape(N_QUERIES // 128, U_DIM, 128)
    return o3.transpose(0, 2, 1).reshape(N_QUERIES, U_DIM)


# speedup vs baseline: 11.4296x; 1.1584x over previous
import functools

import jax
import jax.numpy as jnp
from jax import lax
from jax.experimental import pallas as pl
from jax.experimental.pallas import tpu as pltpu
from jax.experimental.pallas import tpu_sc as plsc

N_SEG = 1_000_000
U_DIM = 8
N_QUERIES = 1_048_576
ROW = 16
M = N_SEG // ROW
U2 = N_SEG * U_DIM // 16
NC, NS, L = 2, 16, 16
NW = NC * NS
QW = N_QUERIES // NW
B = 1024
NB = QW // B
CH = 128
NCH = B // CH

_COARSE_STEPS = 16
_FINE_STEPS = 5


def _body(t_hbm, c_hbm, ts2_hbm, us16_hbm, out_hbm,
          c_v, tq, ridx, rows, gidx, par, urows, obuf, sem):
    wid = lax.axis_index("s") * NC + lax.axis_index("c")
    pltpu.sync_copy(c_hbm, c_v)
    lanes = lax.iota(jnp.int32, L)

    def batch_body(b, _):
        qoff = wid * QW + b * B
        pltpu.sync_copy(t_hbm.at[pl.ds(qoff, B)], tq)

        def vec_coarse(v, _):
            tv = tq[pl.ds(v * L, L)]
            lo = jnp.zeros((L,), jnp.int32)
            hi = jnp.full((L,), M, jnp.int32)

            def step(i, carry):
                lo, hi = carry
                mid = jnp.minimum((lo + hi) >> 1, M - 1)
                le = plsc.load_gather(c_v, [mid]) <= tv
                return jnp.where(le, mid + 1, lo), jnp.where(le, hi, mid)

            lo, hi = lax.fori_loop(0, _COARSE_STEPS, step, (lo, hi))
            ridx[pl.ds(v * L, L)] = jnp.maximum(lo - 1, 0)
            return 0

        lax.fori_loop(0, B // L, vec_coarse, 0)

        descs = [
            pltpu.async_copy(ts2_hbm.at[ridx.at[pl.ds(j * CH, CH)]],
                             rows.at[pl.ds(j * CH, CH)], sem)
            for j in range(NCH)
        ]
        for d in descs:
            d.wait()

        def vec_fine(v, _):
            tv = tq[pl.ds(v * L, L)]
            qv = v * L + lanes
            lo = jnp.zeros((L,), jnp.int32)
            hi = jnp.full((L,), ROW, jnp.int32)

            def step(i, carry):
                lo, hi = carry
                mid = jnp.minimum((lo + hi) >> 1, ROW - 1)
                le = plsc.load_gather(rows, [qv, mid]) <= tv
                return jnp.where(le, mid + 1, lo), jnp.where(le, hi, mid)

            lo, hi = lax.fori_loop(0, _FINE_STEPS, step, (lo, hi))
            rc = ridx[pl.ds(v * L, L)]
            fi = jnp.clip(rc * ROW + lo - 1, 0, N_SEG - 1)
            gidx[pl.ds(v * L, L)] = fi >> 1
            par[pl.ds(v * L, L)] = (fi & 1) << 3
            return 0

        lax.fori_loop(0, B // L, vec_fine, 0)

        descs = [
            pltpu.async_copy(us16_hbm.at[gidx.at[pl.ds(j * CH, CH)]],
                             urows.at[pl.ds(j * CH, CH)], sem)
            for j in range(NCH)
        ]
        for d in descs:
            d.wait()

        def vec_out(v, _):
            qv = v * L + lanes
            pv = par[pl.ds(v * L, L)]
            base = (v >> 3) * 1024 + (v & 7) * L
            for j in range(U_DIM):
                val = plsc.load_gather(urows, [qv, pv + j])
                obuf[pl.ds(base + j * 128, L)] = val
            return 0

        lax.fori_loop(0, B // L, vec_out, 0)
        pltpu.sync_copy(obuf, out_hbm.at[pl.ds(qoff * U_DIM, B * U_DIM)])
        return 0

    lax.fori_loop(0, NB, batch_body, 0)


NBLK = N_SEG // 128
BASE_BLK = NBLK // NW
EXTRA = NBLK - BASE_BLK * NW
TAIL_ROWS = N_SEG - NBLK * 128
TAIL_F = TAIL_ROWS * U_DIM
TAIL_OFF = NBLK * 128 * U_DIM


def _relayout_body(usT_hbm, tail_hbm, out_hbm, ub, ob, tb, sem):
    wid = lax.axis_index("s") * NC + lax.axis_index("c")
    cnt = jnp.where(wid < EXTRA, BASE_BLK + 1, BASE_BLK)
    start = wid * BASE_BLK + jnp.minimum(wid, EXTRA)
    lanes = lax.iota(jnp.int32, L)
    jl = lanes & 7
    col0 = lanes >> 3

    def blk_body(i, _):
        k = start + i
        pltpu.sync_copy(usT_hbm.at[:, pl.ds(k * 128, 128)], ub)

        def vec_body(v, _):
            ob[pl.ds(v * L, L)] = plsc.load_gather(ub, [jl, col0 + 2 * v])
            return 0

        lax.fori_loop(0, 64, vec_body, 0)
        pltpu.sync_copy(ob, out_hbm.at[pl.ds(k * 1024, 1024)])
        return 0

    lax.fori_loop(0, cnt, blk_body, 0)

    @pl.when(wid == NW - 1)
    def _():
        pltpu.sync_copy(tail_hbm, tb)
        pltpu.sync_copy(tb, out_hbm.at[pl.ds(TAIL_OFF, TAIL_F)])


@jax.jit
def _relayout(usT, tail):
    mesh = plsc.VectorSubcoreMesh(core_axis_name="c", subcore_axis_name="s")
    return pl.kernel(
        _relayout_body,
        out_type=jax.ShapeDtypeStruct((N_SEG * U_DIM,), jnp.float32),
        mesh=mesh,
        compiler_params=pltpu.CompilerParams(
            needs_layout_passes=False, use_tc_tiling_on_sc=True),
        scratch_types=[
            pltpu.VMEM((U_DIM, 128), jnp.float32),
            pltpu.VMEM((1024,), jnp.float32),
            pltpu.VMEM((TAIL_F,), jnp.float32),
            pltpu.SemaphoreType.DMA,
        ],
    )(usT, tail)


@jax.jit
def _run(t, c, ts2, us16):
    mesh = plsc.VectorSubcoreMesh(core_axis_name="c", subcore_axis_name="s")
    return pl.kernel(
        _body,
        out_type=jax.ShapeDtypeStruct((N_QUERIES * U_DIM,), jnp.float32),
        mesh=mesh,
        compiler_params=pltpu.CompilerParams(
            needs_layout_passes=False, use_tc_tiling_on_sc=False),
        scratch_types=[
            pltpu.VMEM((M,), jnp.float32),
            pltpu.VMEM((B,), jnp.float32),
            pltpu.VMEM((B,), jnp.int32),
            pltpu.VMEM((B, ROW), jnp.float32),
            pltpu.VMEM((B,), jnp.int32),
            pltpu.VMEM((B,), jnp.int32),
            pltpu.VMEM((B, 16), jnp.float32),
            pltpu.VMEM((B * U_DIM,), jnp.float32),
            pltpu.SemaphoreType.DMA,
        ],
    )(t, c, ts2, us16)


def kernel(t, x, ts, us):
    c = ts[::ROW]
    ts2 = ts.reshape(M, ROW)
    usT = us.T
    tail = us[NBLK * 128:].reshape(TAIL_F)
    us16 = _relayout(usT, tail).reshape(U2, 16)
    o = _run(t, c, ts2, us16)
    o3 = o.reshape(N_QUERIES // 128, U_DIM, 128)
    return o3.transpose(0, 2, 1).reshape(N_QUERIES, U_DIM)


# speedup vs baseline: 13.1987x; 1.1548x over previous
import functools

import jax
import jax.numpy as jnp
from jax import lax
from jax.experimental import pallas as pl
from jax.experimental.pallas import tpu as pltpu
from jax.experimental.pallas import tpu_sc as plsc

N_SEG = 1_000_000
U_DIM = 8
N_QUERIES = 1_048_576
ROW = 16
M = N_SEG // ROW
U2 = N_SEG * U_DIM // 16
NC, NS, L = 2, 16, 16
NW = NC * NS
QW = N_QUERIES // NW
B = 1024
NB = QW // B
CH = 128
NCH = B // CH

_COARSE_STEPS = 16
_FINE_STEPS = 5


def _body(t_hbm, c_hbm, ts2_hbm, us16_hbm, out_hbm,
          c_v, tq, ridx, rows, gidx, par, urows, obuf, sem):
    wid = lax.axis_index("s") * NC + lax.axis_index("c")
    pltpu.sync_copy(c_hbm, c_v)
    lanes = lax.iota(jnp.int32, L)

    def batch_body(b, _):
        qoff = wid * QW + b * B
        pltpu.sync_copy(t_hbm.at[pl.ds(qoff, B)], tq)

        def vec_coarse(v, _):
            tv = tq[pl.ds(v * L, L)]
            lo = jnp.zeros((L,), jnp.int32)
            hi = jnp.full((L,), M, jnp.int32)

            def step(i, carry):
                lo, hi = carry
                mid = jnp.minimum((lo + hi) >> 1, M - 1)
                le = plsc.load_gather(c_v, [mid]) <= tv
                return jnp.where(le, mid + 1, lo), jnp.where(le, hi, mid)

            lo, hi = lax.fori_loop(0, _COARSE_STEPS, step, (lo, hi))
            ridx[pl.ds(v * L, L)] = jnp.maximum(lo - 1, 0)
            return 0

        lax.fori_loop(0, B // L, vec_coarse, 0)

        descs = [
            pltpu.async_copy(ts2_hbm.at[ridx.at[pl.ds(j * CH, CH)]],
                             rows.at[pl.ds(j * CH, CH)], sem)
            for j in range(NCH)
        ]
        for d in descs:
            d.wait()

        def vec_fine(v, _):
            tv = tq[pl.ds(v * L, L)]
            qv = v * L + lanes
            lo = jnp.zeros((L,), jnp.int32)
            hi = jnp.full((L,), ROW, jnp.int32)

            def step(i, carry):
                lo, hi = carry
                mid = jnp.minimum((lo + hi) >> 1, ROW - 1)
                le = plsc.load_gather(rows, [qv, mid]) <= tv
                return jnp.where(le, mid + 1, lo), jnp.where(le, hi, mid)

            lo, hi = lax.fori_loop(0, _FINE_STEPS, step, (lo, hi))
            rc = ridx[pl.ds(v * L, L)]
            fi = jnp.clip(rc * ROW + lo - 1, 0, N_SEG - 1)
            gidx[pl.ds(v * L, L)] = fi >> 1
            par[pl.ds(v * L, L)] = (fi & 1) << 3
            return 0

        lax.fori_loop(0, B // L, vec_fine, 0)

        descs = [
            pltpu.async_copy(us16_hbm.at[gidx.at[pl.ds(j * CH, CH)]],
                             urows.at[pl.ds(j * CH, CH)], sem)
            for j in range(NCH)
        ]
        for d in descs:
            d.wait()

        def vec_out(v, _):
            qv = v * L + lanes
            pv = par[pl.ds(v * L, L)]
            base = (v >> 3) * 1024 + (v & 7) * L
            for j in range(U_DIM):
                val = plsc.load_gather(urows, [qv, pv + j])
                obuf[pl.ds(base + j * 128, L)] = val
            return 0

        lax.fori_loop(0, B // L, vec_out, 0)
        pltpu.sync_copy(obuf, out_hbm.at[pl.ds(qoff * U_DIM, B * U_DIM)])
        return 0

    lax.fori_loop(0, NB, batch_body, 0)


NBLK = N_SEG // 128
BASE_BLK = NBLK // NW
EXTRA = NBLK - BASE_BLK * NW
TAIL_ROWS = N_SEG - NBLK * 128
TAIL_F = TAIL_ROWS * U_DIM
TAIL_OFF = NBLK * 128 * U_DIM


BPC = 32
FULL_CHUNKS = BASE_BLK // BPC
REM_BLK = BASE_BLK - FULL_CHUNKS * BPC


def _relayout_body(usT_hbm, tail_hbm, out_hbm, ub, ob, tb, sem):
    wid = lax.axis_index("s") * NC + lax.axis_index("c")
    start = wid * BASE_BLK
    lanes = lax.iota(jnp.int32, L)
    jl = lanes & 7
    col0 = lanes >> 3

    def chunk(k0, nblk):
        pltpu.sync_copy(usT_hbm.at[:, pl.ds(k0 * 128, nblk * 128)],
                        ub.at[:, pl.ds(0, nblk * 128)])

        def vec_body(v, _):
            kk = v >> 6
            ob[pl.ds(v * L, L)] = plsc.load_gather(
                ub, [jl, col0 + 2 * (v & 63) + (kk << 7)])
            return 0

        lax.fori_loop(0, nblk * 64, vec_body, 0)
        pltpu.sync_copy(ob.at[pl.ds(0, nblk * 1024)],
                        out_hbm.at[pl.ds(k0 * 1024, nblk * 1024)])

    def full_body(i, _):
        chunk(start + i * BPC, BPC)
        return 0

    lax.fori_loop(0, FULL_CHUNKS, full_body, 0)
    chunk(start + FULL_CHUNKS * BPC, REM_BLK)

    @pl.when(wid >= NW - EXTRA)
    def _():
        chunk(NW * BASE_BLK + (wid - (NW - EXTRA)), 1)

    @pl.when(wid == NW - 1)
    def _():
        pltpu.sync_copy(tail_hbm, tb)
        pltpu.sync_copy(tb, out_hbm.at[pl.ds(TAIL_OFF, TAIL_F)])


@jax.jit
def _relayout(usT, tail):
    mesh = plsc.VectorSubcoreMesh(core_axis_name="c", subcore_axis_name="s")
    return pl.kernel(
        _relayout_body,
        out_type=jax.ShapeDtypeStruct((N_SEG * U_DIM,), jnp.float32),
        mesh=mesh,
        compiler_params=pltpu.CompilerParams(
            needs_layout_passes=False, use_tc_tiling_on_sc=True),
        scratch_types=[
            pltpu.VMEM((U_DIM, BPC * 128), jnp.float32),
            pltpu.VMEM((BPC * 1024,), jnp.float32),
            pltpu.VMEM((TAIL_F,), jnp.float32),
            pltpu.SemaphoreType.DMA,
        ],
    )(usT, tail)


@jax.jit
def _run(t, c, ts2, us16):
    mesh = plsc.VectorSubcoreMesh(core_axis_name="c", subcore_axis_name="s")
    return pl.kernel(
        _body,
        out_type=jax.ShapeDtypeStruct((N_QUERIES * U_DIM,), jnp.float32),
        mesh=mesh,
        compiler_params=pltpu.CompilerParams(
            needs_layout_passes=False, use_tc_tiling_on_sc=False),
        scratch_types=[
            pltpu.VMEM((M,), jnp.float32),
            pltpu.VMEM((B,), jnp.float32),
            pltpu.VMEM((B,), jnp.int32),
            pltpu.VMEM((B, ROW), jnp.float32),
            pltpu.VMEM((B,), jnp.int32),
            pltpu.VMEM((B,), jnp.int32),
            pltpu.VMEM((B, 16), jnp.float32),
            pltpu.VMEM((B * U_DIM,), jnp.float32),
            pltpu.SemaphoreType.DMA,
        ],
    )(t, c, ts2, us16)


def kernel(t, x, ts, us):
    c = ts[::ROW]
    ts2 = ts.reshape(M, ROW)
    usT = us.T
    tail = us[NBLK * 128:].reshape(TAIL_F)
    us16 = _relayout(usT, tail).reshape(U2, 16)
    o = _run(t, c, ts2, us16)
    o3 = o.reshape(N_QUERIES // 128, U_DIM, 128)
    return o3.transpose(0, 2, 1).reshape(N_QUERIES, U_DIM)


# speedup vs baseline: 25.1391x; 1.9047x over previous
import functools

import jax
import jax.numpy as jnp
from jax import lax
from jax.experimental import pallas as pl
from jax.experimental.pallas import tpu as pltpu
from jax.experimental.pallas import tpu_sc as plsc

N_SEG = 1_000_000
U_DIM = 8
N_QUERIES = 1_048_576
ROW = 16
M = N_SEG // ROW
U2 = N_SEG * U_DIM // 16
NC, NS, L = 2, 16, 16
NW = NC * NS
QW = N_QUERIES // NW
B = 1024
NB = QW // B
CH = 128
NCH = B // CH

_COARSE_STEPS = 16
_FINE_STEPS = 5


VPC = CH // L
ILP = 4


def _body(t_hbm, c_hbm, ts2_hbm, us16_hbm, out_hbm,
          c_v, tq, ridx, rows, gidx, par, urows, obuf, sem, semg):
    wid = lax.axis_index("s") * NC + lax.axis_index("c")
    pltpu.sync_copy(c_hbm, c_v)
    lanes = lax.iota(jnp.int32, L)

    def batch_body(b, _):
        qoff = wid * QW + b * B
        pltpu.sync_copy(t_hbm.at[pl.ds(qoff, B)], tq)

        def coarse_chunk(ch, _):
            for g in range(VPC // ILP):
                v0 = ch * VPC + g * ILP
                tv = [tq[pl.ds((v0 + i) * L, L)] for i in range(ILP)]
                lo = [jnp.zeros((L,), jnp.int32)] * ILP
                hi = [jnp.full((L,), M, jnp.int32)] * ILP
                for _s in range(_COARSE_STEPS):
                    for i in range(ILP):
                        mid = jnp.minimum((lo[i] + hi[i]) >> 1, M - 1)
                        le = plsc.load_gather(c_v, [mid]) <= tv[i]
                        lo[i] = jnp.where(le, mid + 1, lo[i])
                        hi[i] = jnp.where(le, hi[i], mid)
                for i in range(ILP):
                    ridx[pl.ds((v0 + i) * L, L)] = jnp.maximum(lo[i] - 1, 0)
            pltpu.async_copy(ts2_hbm.at[ridx.at[pl.ds(ch * CH, CH)]],
                             rows.at[pl.ds(ch * CH, CH)], sem.at[ch])
            return 0

        lax.fori_loop(0, NCH, coarse_chunk, 0)

        def fine_chunk(ch, _):
            pltpu.make_async_copy(ts2_hbm.at[ridx.at[pl.ds(ch * CH, CH)]],
                                  rows.at[pl.ds(ch * CH, CH)],
                                  sem.at[ch]).wait()
            for g in range(VPC // ILP):
                v0 = ch * VPC + g * ILP
                tv = [tq[pl.ds((v0 + i) * L, L)] for i in range(ILP)]
                qv = [(v0 + i) * L + lanes for i in range(ILP)]
                lo = [jnp.zeros((L,), jnp.int32)] * ILP
                hi = [jnp.full((L,), ROW, jnp.int32)] * ILP
                for _s in range(_FINE_STEPS):
                    for i in range(ILP):
                        mid = jnp.minimum((lo[i] + hi[i]) >> 1, ROW - 1)
                        le = plsc.load_gather(rows, [qv[i], mid]) <= tv[i]
                        lo[i] = jnp.where(le, mid + 1, lo[i])
                        hi[i] = jnp.where(le, hi[i], mid)
                for i in range(ILP):
                    rc = ridx[pl.ds((v0 + i) * L, L)]
                    fi = jnp.clip(rc * ROW + lo[i] - 1, 0, N_SEG - 1)
                    gidx[pl.ds((v0 + i) * L, L)] = fi >> 1
                    par[pl.ds((v0 + i) * L, L)] = (fi & 1) << 3
            pltpu.async_copy(us16_hbm.at[gidx.at[pl.ds(ch * CH, CH)]],
                             urows.at[pl.ds(ch * CH, CH)], semg.at[ch])
            return 0

        lax.fori_loop(0, NCH, fine_chunk, 0)

        def out_chunk(ch, _):
            pltpu.make_async_copy(us16_hbm.at[gidx.at[pl.ds(ch * CH, CH)]],
                                  urows.at[pl.ds(ch * CH, CH)],
                                  semg.at[ch]).wait()

            def vec_out(v, _):
                qv = v * L + lanes
                pv = par[pl.ds(v * L, L)]
                base = (v >> 3) * 1024 + (v & 7) * L
                for j in range(U_DIM):
                    val = plsc.load_gather(urows, [qv, pv + j])
                    obuf[pl.ds(base + j * 128, L)] = val
                return 0

            lax.fori_loop(ch * VPC, (ch + 1) * VPC, vec_out, 0)
            return 0

        lax.fori_loop(0, NCH, out_chunk, 0)
        pltpu.sync_copy(obuf, out_hbm.at[pl.ds(qoff * U_DIM, B * U_DIM)])
        return 0

    lax.fori_loop(0, NB, batch_body, 0)


NBLK = N_SEG // 128
BASE_BLK = NBLK // NW
EXTRA = NBLK - BASE_BLK * NW
TAIL_ROWS = N_SEG - NBLK * 128
TAIL_F = TAIL_ROWS * U_DIM
TAIL_OFF = NBLK * 128 * U_DIM


BPC = 32
FULL_CHUNKS = BASE_BLK // BPC
REM_BLK = BASE_BLK - FULL_CHUNKS * BPC


def _relayout_body(usT_hbm, tail_hbm, out_hbm, ub, ob, tb, sem):
    wid = lax.axis_index("s") * NC + lax.axis_index("c")
    start = wid * BASE_BLK
    lanes = lax.iota(jnp.int32, L)
    jl = lanes & 7
    col0 = lanes >> 3

    def chunk(k0, nblk):
        pltpu.sync_copy(usT_hbm.at[:, pl.ds(k0 * 128, nblk * 128)],
                        ub.at[:, pl.ds(0, nblk * 128)])

        def vec_body(v, _):
            kk = v >> 6
            ob[pl.ds(v * L, L)] = plsc.load_gather(
                ub, [jl, col0 + 2 * (v & 63) + (kk << 7)])
            return 0

        lax.fori_loop(0, nblk * 64, vec_body, 0)
        pltpu.sync_copy(ob.at[pl.ds(0, nblk * 1024)],
                        out_hbm.at[pl.ds(k0 * 1024, nblk * 1024)])

    def full_body(i, _):
        chunk(start + i * BPC, BPC)
        return 0

    lax.fori_loop(0, FULL_CHUNKS, full_body, 0)
    chunk(start + FULL_CHUNKS * BPC, REM_BLK)

    @pl.when(wid >= NW - EXTRA)
    def _():
        chunk(NW * BASE_BLK + (wid - (NW - EXTRA)), 1)

    @pl.when(wid == NW - 1)
    def _():
        pltpu.sync_copy(tail_hbm, tb)
        pltpu.sync_copy(tb, out_hbm.at[pl.ds(TAIL_OFF, TAIL_F)])


@jax.jit
def _relayout(usT, tail):
    mesh = plsc.VectorSubcoreMesh(core_axis_name="c", subcore_axis_name="s")
    return pl.kernel(
        _relayout_body,
        out_type=jax.ShapeDtypeStruct((N_SEG * U_DIM,), jnp.float32),
        mesh=mesh,
        compiler_params=pltpu.CompilerParams(
            needs_layout_passes=False, use_tc_tiling_on_sc=True),
        scratch_types=[
            pltpu.VMEM((U_DIM, BPC * 128), jnp.float32),
            pltpu.VMEM((BPC * 1024,), jnp.float32),
            pltpu.VMEM((TAIL_F,), jnp.float32),
            pltpu.SemaphoreType.DMA,
        ],
    )(usT, tail)


@jax.jit
def _run(t, c, ts2, us16):
    mesh = plsc.VectorSubcoreMesh(core_axis_name="c", subcore_axis_name="s")
    return pl.kernel(
        _body,
        out_type=jax.ShapeDtypeStruct((N_QUERIES * U_DIM,), jnp.float32),
        mesh=mesh,
        compiler_params=pltpu.CompilerParams(
            needs_layout_passes=False, use_tc_tiling_on_sc=False),
        scratch_types=[
            pltpu.VMEM((M,), jnp.float32),
            pltpu.VMEM((B,), jnp.float32),
            pltpu.VMEM((B,), jnp.int32),
            pltpu.VMEM((B, ROW), jnp.float32),
            pltpu.VMEM((B,), jnp.int32),
            pltpu.VMEM((B,), jnp.int32),
            pltpu.VMEM((B, 16), jnp.float32),
            pltpu.VMEM((B * U_DIM,), jnp.float32),
            pltpu.SemaphoreType.DMA((NCH,)),
            pltpu.SemaphoreType.DMA((NCH,)),
        ],
    )(t, c, ts2, us16)


def kernel(t, x, ts, us):
    c = ts[::ROW]
    ts2 = ts.reshape(M, ROW)
    usT = us.T
    tail = us[NBLK * 128:].reshape(TAIL_F)
    us16 = _relayout(usT, tail).reshape(U2, 16)
    o = _run(t, c, ts2, us16)
    o3 = o.reshape(N_QUERIES // 128, U_DIM, 128)
    return o3.transpose(0, 2, 1).reshape(N_QUERIES, U_DIM)
